# baseline (speedup 1.0000x reference)
import functools

import jax
import jax.numpy as jnp
from jax import lax
from jax.experimental import pallas as pl
from jax.experimental.pallas import tpu as pltpu
from jax.experimental.pallas import tpu_sc as plsc

D_MODEL = 64
GATHER_W = 80
CHUNK = 160
NGATH = CHUNK // GATHER_W


def _embed_lookup(xf, table2, *, n, num_cores, num_subcores):
    nw = num_cores * num_subcores
    per_w = n // nw
    steps = per_w // CHUNK
    assert steps % 2 == 0 and steps >= 6

    mesh = plsc.VectorSubcoreMesh(core_axis_name="c", subcore_axis_name="s")

    @functools.partial(
        pl.kernel,
        mesh=mesh,
        compiler_params=pltpu.CompilerParams(use_tc_tiling_on_sc=True),
        out_type=jax.ShapeDtypeStruct((n, D_MODEL), jnp.float32),
        scratch_types=[
            pltpu.VMEM((CHUNK,), jnp.int32),
            pltpu.VMEM((CHUNK,), jnp.int32),
            pltpu.VMEM((CHUNK, 2 * D_MODEL), jnp.float32),
            pltpu.VMEM((CHUNK, 2 * D_MODEL), jnp.float32),
            pltpu.VMEM((CHUNK, D_MODEL), jnp.float32),
            pltpu.VMEM((CHUNK, D_MODEL), jnp.float32),
            pltpu.SemaphoreType.DMA,
            pltpu.SemaphoreType.DMA,
            pltpu.SemaphoreType.DMA,
            pltpu.SemaphoreType.DMA,
            pltpu.SemaphoreType.DMA,
            pltpu.SemaphoreType.DMA,
        ],
    )
    def k(xf_hbm, t2_hbm, out_hbm, idx0, idx1, rows0, rows1, st0, st1,
          sem_i0, sem_i1, sem_g0, sem_g1, sem_w0, sem_w1):
        idx = (idx0, idx1)
        rows = (rows0, rows1)
        st = (st0, st1)
        sem_i = (sem_i0, sem_i1)
        sem_g = (sem_g0, sem_g1)
        sem_w = (sem_w0, sem_w1)

        wid = lax.axis_index("s") * num_cores + lax.axis_index("c")
        base = wid * per_w

        def drain_idx(g, p):
            pltpu.make_async_copy(
                xf_hbm.at[pl.ds(base + g * CHUNK, CHUNK)], idx[p],
                sem_i[p]).wait()

        def fire_gathers(g, p):
            for j in range(NGATH):
                pltpu.async_copy(
                    t2_hbm.at[idx[p].at[pl.ds(j * GATHER_W, GATHER_W)]],
                    rows[p].at[pl.ds(j * GATHER_W, GATHER_W)],
                    sem_g[p])

        def drain_gathers(p):
            for j in range(NGATH):
                pltpu.make_async_copy(
                    t2_hbm.at[idx[p].at[pl.ds(j * GATHER_W, GATHER_W)]],
                    rows[p].at[pl.ds(j * GATHER_W, GATHER_W)],
                    sem_g[p]).wait()

        def issue_idx(g, p):
            pltpu.async_copy(
                xf_hbm.at[pl.ds(base + g * CHUNK, CHUNK)], idx[p], sem_i[p])

        def depad(p):
            def dep(i, carry):
                for r in range(8):
                    j = i * 8 + r
                    for c in range(D_MODEL // 16):
                        st[p][j, pl.ds(c * 16, 16)] = (
                            rows[p][j, pl.ds(c * 16, 16)])
                return carry

            lax.fori_loop(0, CHUNK // 8, dep, 0)

        def issue_write(g, p):
            pltpu.async_copy(
                st[p], out_hbm.at[pl.ds(base + g * CHUNK, CHUNK)], sem_w[p])

        def drain_write(g, p):
            pltpu.make_async_copy(
                st[p], out_hbm.at[pl.ds(base + g * CHUNK, CHUNK)],
                sem_w[p]).wait()

        def stage(g, p):
            @pl.when(g >= 3)
            def _():
                drain_write(g - 3, 1 - p)
            drain_idx(g, p)
            fire_gathers(g, p)
            drain_gathers(1 - p)
            @pl.when(g + 1 < steps)
            def _():
                issue_idx(g + 1, 1 - p)
            depad(1 - p)
            issue_write(g - 1, 1 - p)

        issue_idx(0, 0)
        drain_idx(0, 0)
        fire_gathers(0, 0)
        issue_idx(1, 1)

        def body(i, carry):
            stage(2 * i + 1, 1)
            stage(2 * i + 2, 0)
            return carry

        lax.fori_loop(0, (steps - 2) // 2, body, 0)

        stage(steps - 1, 1)
        drain_write(steps - 3, 1)
        drain_gathers(1)
        depad(1)
        issue_write(steps - 1, 1)
        drain_write(steps - 2, 0)
        drain_write(steps - 1, 1)

    return k(xf, table2)


PAD_C = 160


def _pad_table(table, *, num_cores, num_subcores):
    v = table.shape[0]
    nw = num_cores * num_subcores
    chunks = v // PAD_C
    nk = (chunks + nw - 1) // nw
    if nk % 2 == 1:
        nk += 1

    mesh = plsc.VectorSubcoreMesh(core_axis_name="c", subcore_axis_name="s")

    @functools.partial(
        pl.kernel,
        mesh=mesh,
        compiler_params=pltpu.CompilerParams(use_tc_tiling_on_sc=True),
        out_type=jax.ShapeDtypeStruct((v, 2 * D_MODEL), jnp.float32),
        scratch_types=[
            pltpu.VMEM((PAD_C, D_MODEL), jnp.float32),
            pltpu.VMEM((PAD_C, D_MODEL), jnp.float32),
            pltpu.VMEM((PAD_C, 2 * D_MODEL), jnp.float32),
            pltpu.VMEM((PAD_C, 2 * D_MODEL), jnp.float32),
            pltpu.SemaphoreType.DMA,
            pltpu.SemaphoreType.DMA,
            pltpu.SemaphoreType.DMA,
            pltpu.SemaphoreType.DMA,
        ],
    )
    def kp(t_hbm, out_hbm, a0, a1, b0, b1, sem_a0, sem_a1, sem_b0, sem_b1):
        a = (a0, a1)
        bb = (b0, b1)
        sem_a = (sem_a0, sem_a1)
        sem_b = (sem_b0, sem_b1)

        wid = lax.axis_index("s") * num_cores + lax.axis_index("c")

        def cid_of(k):
            return k * nw + wid

        def issue_in(k, p):
            pltpu.async_copy(
                t_hbm.at[pl.ds(cid_of(k) * PAD_C, PAD_C)], a[p], sem_a[p])

        def drain_in(k, p):
            pltpu.make_async_copy(
                t_hbm.at[pl.ds(cid_of(k) * PAD_C, PAD_C)], a[p],
                sem_a[p]).wait()

        def issue_out(k, p):
            pltpu.async_copy(
                bb[p], out_hbm.at[pl.ds(cid_of(k) * PAD_C, PAD_C)], sem_b[p])

        def drain_out(k, p):
            pltpu.make_async_copy(
                bb[p], out_hbm.at[pl.ds(cid_of(k) * PAD_C, PAD_C)],
                sem_b[p]).wait()

        def repack(p):
            def rep(i, carry):
                for r in range(8):
                    j = i * 8 + r
                    for c in range(D_MODEL // 16):
                        bb[p][j, pl.ds(c * 16, 16)] = (
                            a[p][j, pl.ds(c * 16, 16)])
                return carry

            lax.fori_loop(0, PAD_C // 8, rep, 0)

        def valid(k):
            return cid_of(k) < chunks

        def substep(k, p):
            @pl.when(valid(k))
            def _():
                drain_in(k, p)
            @pl.when((k >= 2) & valid(k - 2))
            def _():
                drain_out(k - 2, p)
            @pl.when(valid(k))
            def _():
                repack(p)
                issue_out(k, p)
            @pl.when(valid(k + 2))
            def _():
                issue_in(k + 2, p)

        @pl.when(valid(0))
        def _():
            issue_in(0, 0)

        @pl.when(valid(1))
        def _():
            issue_in(1, 1)

        def body(i, carry):
            substep(2 * i, 0)
            substep(2 * i + 1, 1)
            return carry

        lax.fori_loop(0, nk // 2, body, 0)

        for p in range(2):
            for k in range(nk - 2, nk):
                if k % 2 == p:
                    @pl.when(valid(k))
                    def _():
                        drain_out(k, p)

    return kp(table)


def kernel(x, table):
    b, l = x.shape
    n = b * l
    info = plsc.get_sparse_core_info()
    xf = x.reshape(n)
    table2 = _pad_table(
        table, num_cores=info.num_cores, num_subcores=info.num_subcores)
    out = _embed_lookup(
        xf, table2, n=n,
        num_cores=info.num_cores, num_subcores=info.num_subcores,
    )
    return out.reshape(b, l, D_MODEL)


# speedup vs baseline: 1.1605x; 1.1605x over previous
import functools

import jax
import jax.numpy as jnp
from jax import lax
from jax.experimental import pallas as pl
from jax.experimental.pallas import tpu as pltpu
from jax.experimental.pallas import tpu_sc as plsc

D_MODEL = 64
GATHER_W = 80
CHUNK = 160
NGATH = CHUNK // GATHER_W


def _embed_lookup(xf, table2, *, n, num_cores, num_subcores):
    nw = num_cores * num_subcores
    per_w = n // nw
    steps = per_w // CHUNK
    assert steps % 2 == 0 and steps >= 6

    mesh = plsc.VectorSubcoreMesh(core_axis_name="c", subcore_axis_name="s")

    @functools.partial(
        pl.kernel,
        mesh=mesh,
        compiler_params=pltpu.CompilerParams(use_tc_tiling_on_sc=True),
        out_type=jax.ShapeDtypeStruct((n, D_MODEL), jnp.float32),
        scratch_types=[
            pltpu.VMEM((CHUNK,), jnp.int32),
            pltpu.VMEM((CHUNK,), jnp.int32),
            pltpu.VMEM((CHUNK, 2 * D_MODEL), jnp.float32),
            pltpu.VMEM((CHUNK, 2 * D_MODEL), jnp.float32),
            pltpu.VMEM((CHUNK, D_MODEL), jnp.float32),
            pltpu.VMEM((CHUNK, D_MODEL), jnp.float32),
            pltpu.SemaphoreType.DMA,
            pltpu.SemaphoreType.DMA,
            pltpu.SemaphoreType.DMA,
            pltpu.SemaphoreType.DMA,
            pltpu.SemaphoreType.DMA,
            pltpu.SemaphoreType.DMA,
        ],
    )
    def k(xf_hbm, t2_hbm, out_hbm, idx0, idx1, rows0, rows1, st0, st1,
          sem_i0, sem_i1, sem_g0, sem_g1, sem_w0, sem_w1):
        idx = (idx0, idx1)
        rows = (rows0, rows1)
        st = (st0, st1)
        sem_i = (sem_i0, sem_i1)
        sem_g = (sem_g0, sem_g1)
        sem_w = (sem_w0, sem_w1)

        wid = lax.axis_index("s") * num_cores + lax.axis_index("c")
        base = wid * per_w

        def drain_idx(g, p):
            pltpu.make_async_copy(
                xf_hbm.at[pl.ds(base + g * CHUNK, CHUNK)], idx[p],
                sem_i[p]).wait()

        def fire_gathers(g, p):
            for j in range(NGATH):
                pltpu.async_copy(
                    t2_hbm.at[idx[p].at[pl.ds(j * GATHER_W, GATHER_W)]],
                    rows[p].at[pl.ds(j * GATHER_W, GATHER_W)],
                    sem_g[p])

        def drain_gathers(p):
            for j in range(NGATH):
                pltpu.make_async_copy(
                    t2_hbm.at[idx[p].at[pl.ds(j * GATHER_W, GATHER_W)]],
                    rows[p].at[pl.ds(j * GATHER_W, GATHER_W)],
                    sem_g[p]).wait()

        def issue_idx(g, p):
            pltpu.async_copy(
                xf_hbm.at[pl.ds(base + g * CHUNK, CHUNK)], idx[p], sem_i[p])

        def depad(p):
            def dep(i, carry):
                for r in range(8):
                    j = i * 8 + r
                    for c in range(D_MODEL // 16):
                        st[p][j, pl.ds(c * 16, 16)] = (
                            rows[p][j, pl.ds(c * 16, 16)])
                return carry

            lax.fori_loop(0, CHUNK // 8, dep, 0)

        def issue_write(g, p):
            pltpu.async_copy(
                st[p], out_hbm.at[pl.ds(base + g * CHUNK, CHUNK)], sem_w[p])

        def drain_write(g, p):
            pltpu.make_async_copy(
                st[p], out_hbm.at[pl.ds(base + g * CHUNK, CHUNK)],
                sem_w[p]).wait()

        def stage(g, p):
            @pl.when(g >= 3)
            def _():
                drain_write(g - 3, 1 - p)
            drain_idx(g, p)
            fire_gathers(g, p)
            drain_gathers(1 - p)
            @pl.when(g + 1 < steps)
            def _():
                issue_idx(g + 1, 1 - p)
            depad(1 - p)
            issue_write(g - 1, 1 - p)

        issue_idx(0, 0)
        drain_idx(0, 0)
        fire_gathers(0, 0)
        issue_idx(1, 1)

        def body(i, carry):
            stage(2 * i + 1, 1)
            stage(2 * i + 2, 0)
            return carry

        lax.fori_loop(0, (steps - 2) // 2, body, 0)

        stage(steps - 1, 1)
        drain_write(steps - 3, 1)
        drain_gathers(1)
        depad(1)
        issue_write(steps - 1, 1)
        drain_write(steps - 2, 0)
        drain_write(steps - 1, 1)

    return k(xf, table2)


def kernel(x, table):
    b, l = x.shape
    n = b * l
    info = plsc.get_sparse_core_info()
    xf = x.reshape(n)
    table2 = jnp.pad(table, ((0, 0), (0, table.shape[1])))
    out = _embed_lookup(
        xf, table2, n=n,
        num_cores=info.num_cores, num_subcores=info.num_subcores,
    )
    return out.reshape(b, l, D_MODEL)
